# Initial kernel scaffold; baseline (speedup 1.0000x reference)
import jax
import jax.numpy as jnp
from jax.experimental import pallas as pl


def kernel(pred_beta, pred_ccoords, pred_energy, pred_pos, pred_time, pred_id, t_idx, t_energy, t_pos, t_time, t_pid, rowsplits):
    raise NotImplementedError("write your pallas kernel here")


# speedup vs baseline: 2.3198x; 2.3198x over previous
import functools

import jax
import jax.numpy as jnp
from jax.experimental import pallas as pl
from jax.experimental.pallas import tpu as pltpu

V = 50000
K = 256
BH = 512
NB = 98
VP = BH * NB
Q_MIN = 0.5
S_B = 1.0


def _atanh(b):
    return 0.5 * jnp.log((1.0 + b) / (1.0 - b))


def _body(feat_ref, sidx_ref, out_ref, maxb_s, sums_s, alpha_s, scal_s):
    p = pl.program_id(0)
    b = pl.program_id(1)

    s = sidx_ref[...]
    hid = b * BH + jax.lax.broadcasted_iota(jnp.int32, (BH, 1), 0)
    valid = hid < V
    nn = jnp.logical_and(s >= 0, valid)
    nnf = nn.astype(jnp.float32)
    sc = jnp.clip(s, 0, K - 1)
    kiota = jax.lax.broadcasted_iota(jnp.int32, (BH, K), 1)
    onehot = jnp.logical_and(sc == kiota, nn)
    beta = jnp.clip(feat_ref[:, 0:1], 1e-4, 1.0 - 1e-4)

    @pl.when(jnp.logical_and(p == 0, b == 0))
    def _init():
        maxb_s[...] = jnp.full((8, K), -1.0, jnp.float32)
        sums_s[...] = jnp.zeros((16, K), jnp.float32)
        for i in range(8):
            scal_s[i] = 0.0

    @pl.when(p == 0)
    def _phase0():
        cand = jnp.where(onehot, beta, -1.0)
        bm = jnp.max(cand, axis=0, keepdims=True)
        maxb_s[...] = jnp.maximum(maxb_s[...], jnp.broadcast_to(bm, (8, K)))

    @pl.when(p == 1)
    def _phase1():
        x0 = feat_ref[:, 1:2]
        x1 = feat_ref[:, 2:3]
        pe = feat_ref[:, 3:4]
        te = feat_ref[:, 4:5]
        pp0 = feat_ref[:, 5:6]
        pp1 = feat_ref[:, 6:7]
        tp0 = feat_ref[:, 7:8]
        tp1 = feat_ref[:, 8:9]
        tt = feat_ref[:, 9:10]
        q = (_atanh(beta) ** 2 + Q_MIN) * tt
        vm = valid.astype(jnp.float32)
        pad1 = 1.0 - vm
        el = (te - pe) ** 2 / (te * te + pad1)
        pll = ((tp0 - pp0) ** 2 / (tp0 * tp0 + pad1)
               + (tp1 - pp1) ** 2 / (tp1 * tp1 + pad1))
        maxb_hit = jnp.sum(
            jnp.where(onehot, maxb_s[0:1, :], 0.0), axis=1, keepdims=True
        )
        is_a = (beta == maxb_hit).astype(jnp.float32) * nnf
        pw = beta * nnf
        W = jnp.concatenate(
            [nnf, is_a, is_a * x0, is_a * x1, is_a * q, is_a * beta,
             pw, pw * el, pw * pll, jnp.zeros((BH, 7), jnp.float32)],
            axis=1,
        )
        onehotf = onehot.astype(jnp.float32)
        part = jax.lax.dot_general(
            W, onehotf, (((0,), (0,)), ((), ())),
            precision=jax.lax.Precision.HIGHEST,
            preferred_element_type=jnp.float32,
        )
        sums_s[...] = sums_s[...] + part
        isn = vm * (1.0 - nnf)
        scal_s[2] = scal_s[2] + jnp.sum(beta * isn)
        scal_s[3] = scal_s[3] + jnp.sum(isn)
        idv = feat_ref[:, 10:16]
        scal_s[4] = scal_s[4] + jnp.sum(idv * idv)

    @pl.when(jnp.logical_and(p == 2, b == 0))
    def _epilogue():
        nh = sums_s[0:1, :]
        den = sums_s[1:2, :] + 1e-9
        xa0 = sums_s[2:3, :] / den
        xa1 = sums_s[3:4, :] / den
        qa = sums_s[4:5, :] / den
        ba = sums_s[5:6, :] / den
        exists = (nh > 0.0).astype(jnp.float32)
        plden = sums_s[6:7, :] + 1e-9
        pl0 = sums_s[7:8, :] / plden
        pl1 = sums_s[8:9, :] / plden
        alpha_s[...] = jnp.concatenate(
            [xa0, xa1, qa, exists, jnp.zeros((4, K), jnp.float32)], axis=0
        )
        n_obj = jnp.sum(exists) + 1e-9
        minb = jnp.sum((1.0 - ba) * exists) / n_obj
        payload = jnp.sum((pl0 + pl1) * exists) / n_obj
        scal_s[5] = minb + payload

    @pl.when(p == 2)
    def _phase2():
        x0 = feat_ref[:, 1:2]
        x1 = feat_ref[:, 2:3]
        tt = feat_ref[:, 9:10]
        q = (_atanh(beta) ** 2 + Q_MIN) * tt
        xa0r = alpha_s[0:1, :]
        xa1r = alpha_s[1:2, :]
        qar = alpha_s[2:3, :]
        exr = alpha_s[3:4, :]
        xa0h = jnp.sum(jnp.where(onehot, xa0r, 0.0), axis=1, keepdims=True)
        xa1h = jnp.sum(jnp.where(onehot, xa1r, 0.0), axis=1, keepdims=True)
        qah = jnp.sum(jnp.where(onehot, qar, 0.0), axis=1, keepdims=True)
        att_p = jnp.sum(
            q * qah * ((x0 - xa0h) ** 2 + (x1 - xa1h) ** 2) * nnf
        )
        d2 = (x0 - xa0r) ** 2 + (x1 - xa1r) ** 2
        d = jnp.sqrt(d2 + 1e-9)
        hinge = jnp.maximum(0.0, 1.0 - d)
        onehotf = onehot.astype(jnp.float32)
        repm = hinge * q * qar * (1.0 - onehotf) * exr * nnf
        rep_p = jnp.sum(repm)
        scal_s[0] = scal_s[0] + att_p
        scal_s[1] = scal_s[1] + rep_p

        @pl.when(b == NB - 1)
        def _final():
            noise = S_B * scal_s[2] / (scal_s[3] + 1e-9)
            cls = 1e-8 * scal_s[4] / (V * 6.0)
            loss = (scal_s[0] / V + scal_s[1] / V + scal_s[5] + noise + cls)
            out_ref[...] = jnp.reshape(loss, (1, 1))


@functools.partial(jax.jit, static_argnames=("interpret",))
def _run(feat_p, sidx_p, interpret=False):
    return pl.pallas_call(
        _body,
        grid=(3, NB),
        in_specs=[
            pl.BlockSpec((BH, 16), lambda p, b: (b, 0)),
            pl.BlockSpec((BH, 1), lambda p, b: (b, 0)),
        ],
        out_specs=pl.BlockSpec((1, 1), lambda p, b: (0, 0)),
        out_shape=jax.ShapeDtypeStruct((1, 1), jnp.float32),
        scratch_shapes=[
            pltpu.VMEM((8, K), jnp.float32),
            pltpu.VMEM((16, K), jnp.float32),
            pltpu.VMEM((8, K), jnp.float32),
            pltpu.SMEM((8,), jnp.float32),
        ],
        interpret=interpret,
    )(feat_p, sidx_p)


def kernel(pred_beta, pred_ccoords, pred_energy, pred_pos, pred_time,
           pred_id, t_idx, t_energy, t_pos, t_time, t_pid, rowsplits):
    feat = jnp.concatenate(
        [pred_beta, pred_ccoords, pred_energy, t_energy, pred_pos, t_pos,
         t_time, pred_id], axis=1)
    pad = VP - V
    feat_p = jnp.pad(feat, ((0, pad), (0, 0)))
    sidx_p = jnp.pad(t_idx, ((0, pad), (0, 0)), constant_values=-1)
    loss = _run(feat_p, sidx_p)
    return pred_beta, jnp.reshape(loss, (1,))


# speedup vs baseline: 2.7483x; 1.1847x over previous
import functools

import jax
import jax.numpy as jnp
from jax.experimental import pallas as pl
from jax.experimental.pallas import tpu as pltpu

V = 50000
K = 256
BH = 1024
NB = 49
VP = BH * NB
Q_MIN = 0.5
S_B = 1.0
HI = jax.lax.Precision.HIGHEST


def _atanh(b):
    return 0.5 * jnp.log((1.0 + b) / (1.0 - b))


def _r8(x):
    return jnp.sum(x.reshape(BH // 8, 8, x.shape[1]), axis=0)


def _gather(onehotf, table):
    return jax.lax.dot_general(
        onehotf, table, (((1,), (1,)), ((), ())),
        precision=HI, preferred_element_type=jnp.float32)


def _body(feat_ref, sidx_ref, out_ref, maxb_s, sums_s, alpha_s, repacc_s,
          sacc_s, scal_s):
    p = pl.program_id(0)
    b = pl.program_id(1)

    s = sidx_ref[...]
    hid = b * BH + jax.lax.broadcasted_iota(jnp.int32, (BH, 1), 0)
    valid = hid < V
    nn = jnp.logical_and(s >= 0, valid)
    nnf = nn.astype(jnp.float32)
    sc = jnp.clip(s, 0, K - 1)
    kiota = jax.lax.broadcasted_iota(jnp.int32, (BH, K), 1)
    onehot = jnp.logical_and(sc == kiota, nn)
    beta = jnp.clip(feat_ref[:, 0:1], 1e-4, 1.0 - 1e-4)

    @pl.when(jnp.logical_and(p == 0, b == 0))
    def _init():
        maxb_s[...] = jnp.full((8, K), -1.0, jnp.float32)
        sums_s[...] = jnp.zeros((16, K), jnp.float32)
        repacc_s[...] = jnp.zeros((8, K), jnp.float32)
        sacc_s[...] = jnp.zeros((8, 128), jnp.float32)

    @pl.when(p == 0)
    def _phase0():
        cand = jnp.where(onehot, beta, -1.0)
        bm8 = jnp.max(cand.reshape(BH // 8, 8, K), axis=0)
        maxb_s[...] = jnp.maximum(maxb_s[...], bm8)

    @pl.when(jnp.logical_and(p == 1, b == 0))
    def _collapse_max():
        m = jnp.max(maxb_s[...], axis=0, keepdims=True)
        maxb_s[...] = jnp.broadcast_to(m, (8, K))

    @pl.when(p == 1)
    def _phase1():
        x0 = feat_ref[:, 1:2]
        x1 = feat_ref[:, 2:3]
        pe = feat_ref[:, 3:4]
        te = feat_ref[:, 4:5]
        pp0 = feat_ref[:, 5:6]
        pp1 = feat_ref[:, 6:7]
        tp0 = feat_ref[:, 7:8]
        tp1 = feat_ref[:, 8:9]
        tt = feat_ref[:, 9:10]
        q = (_atanh(beta) ** 2 + Q_MIN) * tt
        vm = valid.astype(jnp.float32)
        pad1 = 1.0 - vm
        el = (te - pe) ** 2 / (te * te + pad1)
        pll = ((tp0 - pp0) ** 2 / (tp0 * tp0 + pad1)
               + (tp1 - pp1) ** 2 / (tp1 * tp1 + pad1))
        onehotf = onehot.astype(jnp.float32)
        maxb_hit = _gather(onehotf, maxb_s[...])[:, 0:1]
        is_a = (beta == maxb_hit).astype(jnp.float32) * nnf
        pw = beta * nnf
        W = jnp.concatenate(
            [nnf, is_a, is_a * x0, is_a * x1, is_a * q, is_a * beta,
             pw, pw * el, pw * pll, jnp.zeros((BH, 7), jnp.float32)],
            axis=1,
        )
        part = jax.lax.dot_general(
            W, onehotf, (((0,), (0,)), ((), ())),
            precision=HI, preferred_element_type=jnp.float32)
        sums_s[...] = sums_s[...] + part
        isn = vm * (1.0 - nnf)
        idv = feat_ref[:, 10:16]
        spart = jnp.concatenate([beta * isn, isn, idv * idv], axis=1)
        sacc_s[:, 0:8] = sacc_s[:, 0:8] + _r8(spart)

    @pl.when(jnp.logical_and(p == 2, b == 0))
    def _epilogue():
        nh = sums_s[0:1, :]
        den = sums_s[1:2, :] + 1e-9
        xa0 = sums_s[2:3, :] / den
        xa1 = sums_s[3:4, :] / den
        qa = sums_s[4:5, :] / den
        ba = sums_s[5:6, :] / den
        exists = (nh > 0.0).astype(jnp.float32)
        plden = sums_s[6:7, :] + 1e-9
        pl0 = sums_s[7:8, :] / plden
        pl1 = sums_s[8:9, :] / plden
        wk = qa * exists
        alpha_s[...] = jnp.concatenate(
            [xa0, xa1, qa, wk, jnp.zeros((4, K), jnp.float32)], axis=0)
        n_obj = jnp.sum(exists) + 1e-9
        minb = jnp.sum((1.0 - ba) * exists) / n_obj
        payload = jnp.sum((pl0 + pl1) * exists) / n_obj
        scal_s[5] = minb + payload

    @pl.when(p == 2)
    def _phase2():
        x0 = feat_ref[:, 1:2]
        x1 = feat_ref[:, 2:3]
        tt = feat_ref[:, 9:10]
        q = (_atanh(beta) ** 2 + Q_MIN) * tt
        qb = q * nnf
        onehotf = onehot.astype(jnp.float32)
        g = _gather(onehotf, alpha_s[...])
        xa0h = g[:, 0:1]
        xa1h = g[:, 1:2]
        qah = g[:, 2:3]
        wkh = g[:, 3:4]
        d2a = (x0 - xa0h) ** 2 + (x1 - xa1h) ** 2
        att_h = qb * qah * d2a
        hs = jnp.maximum(0.0, 1.0 - jnp.sqrt(d2a + 1e-9))
        same_h = hs * qb * wkh
        xa0r = alpha_s[0:1, :]
        xa1r = alpha_s[1:2, :]
        wkr = alpha_s[3:4, :]
        d2 = (x0 - xa0r) ** 2 + (x1 - xa1r) ** 2
        hinge = jnp.maximum(0.0, 1.0 - jnp.sqrt(d2 + 1e-9))
        repm = hinge * qb * wkr
        repacc_s[...] = repacc_s[...] + _r8(repm)
        sacc_s[:, 8:10] = sacc_s[:, 8:10] + _r8(
            jnp.concatenate([att_h, same_h], axis=1))

        @pl.when(b == NB - 1)
        def _final():
            att = jnp.sum(sacc_s[:, 8:9])
            corr = jnp.sum(sacc_s[:, 9:10])
            rep = jnp.sum(repacc_s[...]) - corr
            noise_num = jnp.sum(sacc_s[:, 0:1])
            noise_den = jnp.sum(sacc_s[:, 1:2])
            idsq = jnp.sum(sacc_s[:, 2:8])
            noise = S_B * noise_num / (noise_den + 1e-9)
            cls = 1e-8 * idsq / (V * 6.0)
            loss = att / V + rep / V + scal_s[5] + noise + cls
            out_ref[...] = jnp.reshape(loss, (1, 1))


@functools.partial(jax.jit, static_argnames=("interpret",))
def _run(feat_p, sidx_p, interpret=False):
    return pl.pallas_call(
        _body,
        grid=(3, NB),
        in_specs=[
            pl.BlockSpec((BH, 16), lambda p, b: (b, 0)),
            pl.BlockSpec((BH, 1), lambda p, b: (b, 0)),
        ],
        out_specs=pl.BlockSpec((1, 1), lambda p, b: (0, 0)),
        out_shape=jax.ShapeDtypeStruct((1, 1), jnp.float32),
        scratch_shapes=[
            pltpu.VMEM((8, K), jnp.float32),
            pltpu.VMEM((16, K), jnp.float32),
            pltpu.VMEM((8, K), jnp.float32),
            pltpu.VMEM((8, K), jnp.float32),
            pltpu.VMEM((8, 128), jnp.float32),
            pltpu.SMEM((8,), jnp.float32),
        ],
        interpret=interpret,
    )(feat_p, sidx_p)


def kernel(pred_beta, pred_ccoords, pred_energy, pred_pos, pred_time,
           pred_id, t_idx, t_energy, t_pos, t_time, t_pid, rowsplits):
    feat = jnp.concatenate(
        [pred_beta, pred_ccoords, pred_energy, t_energy, pred_pos, t_pos,
         t_time, pred_id], axis=1)
    pad = VP - V
    feat_p = jnp.pad(feat, ((0, pad), (0, 0)))
    sidx_p = jnp.pad(t_idx, ((0, pad), (0, 0)), constant_values=-1)
    loss = _run(feat_p, sidx_p)
    return pred_beta, jnp.reshape(loss, (1,))


# speedup vs baseline: 3.8108x; 1.3866x over previous
import functools

import jax
import jax.numpy as jnp
from jax.experimental import pallas as pl
from jax.experimental.pallas import tpu as pltpu

V = 50000
K = 256
BH = 1024
NB = 49
VP = BH * NB
Q_MIN = 0.5
S_B = 1.0
HI = jax.lax.Precision.HIGHEST


def _atanh(b):
    return 0.5 * jnp.log((1.0 + b) / (1.0 - b))


def _nat(a, b):
    return jax.lax.dot_general(a, b, (((1,), (0,)), ((), ())),
                               precision=HI,
                               preferred_element_type=jnp.float32)


def _body(featT_ref, sidxC_ref, sidxR_ref, betaC_ref, out_ref,
          maxb_s, sums_s, alpha_s, alphaT_s, repacc_s, sacc_s, scal_s):
    p = pl.program_id(0)
    b = pl.program_id(1)

    @pl.when(jnp.logical_and(p == 0, b == 0))
    def _init():
        maxb_s[...] = jnp.full((8, K), -1.0, jnp.float32)
        sums_s[...] = jnp.zeros((16, K), jnp.float32)
        repacc_s[...] = jnp.zeros((8, BH), jnp.float32)
        sacc_s[...] = jnp.zeros((8, BH), jnp.float32)

    def col_onehot():
        s = sidxC_ref[...]
        hid = b * BH + jax.lax.broadcasted_iota(jnp.int32, (BH, 1), 0)
        nn = jnp.logical_and(s >= 0, hid < V)
        sc = jnp.clip(s, 0, K - 1)
        kiota = jax.lax.broadcasted_iota(jnp.int32, (BH, K), 1)
        return jnp.logical_and(sc == kiota, nn)

    def row_masks():
        s = sidxR_ref[0]
        hid = b * BH + jax.lax.broadcasted_iota(jnp.int32, (1, BH), 1)
        validb = hid < V
        nnb = jnp.logical_and(s >= 0, validb)
        sc = jnp.clip(s, 0, K - 1)
        kiota = jax.lax.broadcasted_iota(jnp.int32, (K, BH), 0)
        onehotT = jnp.logical_and(sc == kiota, nnb)
        return (validb.astype(jnp.float32), nnb.astype(jnp.float32),
                onehotT.astype(jnp.float32))

    @pl.when(p == 0)
    def _phase0():
        onehot = col_onehot()
        beta = jnp.clip(betaC_ref[...], 1e-4, 1.0 - 1e-4)
        cand = jnp.where(onehot, beta, -1.0)
        bm8 = jnp.max(cand.reshape(BH // 8, 8, K), axis=0)
        maxb_s[...] = jnp.maximum(maxb_s[...], bm8)

    @pl.when(jnp.logical_and(p == 1, b == 0))
    def _collapse_max():
        m = jnp.max(maxb_s[...], axis=0, keepdims=True)
        maxb_s[...] = jnp.broadcast_to(m, (8, K))

    @pl.when(p == 1)
    def _phase1():
        vm, nn, onehotT = row_masks()
        beta = jnp.clip(featT_ref[0:1, :], 1e-4, 1.0 - 1e-4)
        x0 = featT_ref[1:2, :]
        x1 = featT_ref[2:3, :]
        pe = featT_ref[3:4, :]
        te = featT_ref[4:5, :]
        pp0 = featT_ref[5:6, :]
        pp1 = featT_ref[6:7, :]
        tp0 = featT_ref[7:8, :]
        tp1 = featT_ref[8:9, :]
        tt = featT_ref[9:10, :]
        q = (_atanh(beta) ** 2 + Q_MIN) * tt
        pad1 = 1.0 - vm
        el = (te - pe) ** 2 / (te * te + pad1)
        pll = ((tp0 - pp0) ** 2 / (tp0 * tp0 + pad1)
               + (tp1 - pp1) ** 2 / (tp1 * tp1 + pad1))
        gmax = _nat(maxb_s[...], onehotT)
        maxb_hit = gmax[0:1, :]
        is_a = (beta == maxb_hit).astype(jnp.float32) * nn
        pw = beta * nn
        Wt = jnp.concatenate(
            [nn, is_a, is_a * x0, is_a * x1, is_a * q, is_a * beta,
             pw, pw * el, pw * pll, jnp.zeros((7, BH), jnp.float32)],
            axis=0)
        onehotf = col_onehot().astype(jnp.float32)
        sums_s[...] = sums_s[...] + _nat(Wt, onehotf)
        isn = vm * (1.0 - nn)
        idv = featT_ref[10:16, :]
        idsq = jnp.sum(idv * idv, axis=0, keepdims=True)
        sacc_s[...] = sacc_s[...] + jnp.concatenate(
            [beta * isn, isn, idsq, jnp.zeros((5, BH), jnp.float32)],
            axis=0)

    @pl.when(jnp.logical_and(p == 2, b == 0))
    def _epilogue():
        nh = sums_s[0:1, :]
        den = sums_s[1:2, :] + 1e-9
        xa0 = sums_s[2:3, :] / den
        xa1 = sums_s[3:4, :] / den
        qa = sums_s[4:5, :] / den
        ba = sums_s[5:6, :] / den
        exists = (nh > 0.0).astype(jnp.float32)
        plden = sums_s[6:7, :] + 1e-9
        pl0 = sums_s[7:8, :] / plden
        pl1 = sums_s[8:9, :] / plden
        wk = qa * exists
        arows = jnp.concatenate(
            [xa0, xa1, qa, wk, jnp.zeros((4, K), jnp.float32)], axis=0)
        alpha_s[...] = arows
        alphaT_s[...] = jax.lax.transpose(arows, (1, 0))
        n_obj = jnp.sum(exists) + 1e-9
        minb = jnp.sum((1.0 - ba) * exists) / n_obj
        payload = jnp.sum((pl0 + pl1) * exists) / n_obj
        scal_s[5] = minb + payload

    @pl.when(p == 2)
    def _phase2():
        vm, nn, onehotT = row_masks()
        beta = jnp.clip(featT_ref[0:1, :], 1e-4, 1.0 - 1e-4)
        x0 = featT_ref[1:2, :]
        x1 = featT_ref[2:3, :]
        tt = featT_ref[9:10, :]
        q = (_atanh(beta) ** 2 + Q_MIN) * tt
        qb = q * nn
        gT = _nat(alpha_s[...], onehotT)
        xa0h = gT[0:1, :]
        xa1h = gT[1:2, :]
        qah = gT[2:3, :]
        wkh = gT[3:4, :]
        d2a = (x0 - xa0h) ** 2 + (x1 - xa1h) ** 2
        att_r = (qb * qah) * d2a
        hs = jnp.maximum(0.0, 1.0 - jnp.sqrt(d2a + 1e-9))
        same_r = (hs * qb) * wkh
        xa0c = alphaT_s[:, 0:1]
        xa1c = alphaT_s[:, 1:2]
        wkc = alphaT_s[:, 3:4]
        d2 = (x0 - xa0c) ** 2 + (x1 - xa1c) ** 2
        hinge = jnp.maximum(0.0, 1.0 - jnp.sqrt(d2 + 1e-9))
        repm = (hinge * qb) * wkc
        repacc_s[...] = repacc_s[...] + jnp.sum(
            repm.reshape(K // 8, 8, BH), axis=0)
        sacc_s[...] = sacc_s[...] + jnp.concatenate(
            [jnp.zeros((3, BH), jnp.float32), att_r, same_r,
             jnp.zeros((3, BH), jnp.float32)], axis=0)

        @pl.when(b == NB - 1)
        def _final():
            att = jnp.sum(sacc_s[3:4, :])
            corr = jnp.sum(sacc_s[4:5, :])
            rep = jnp.sum(repacc_s[...]) - corr
            noise_num = jnp.sum(sacc_s[0:1, :])
            noise_den = jnp.sum(sacc_s[1:2, :])
            idsq = jnp.sum(sacc_s[2:3, :])
            noise = S_B * noise_num / (noise_den + 1e-9)
            cls = 1e-8 * idsq / (V * 6.0)
            loss = att / V + rep / V + scal_s[5] + noise + cls
            out_ref[...] = jnp.reshape(loss, (1, 1))


@functools.partial(jax.jit, static_argnames=("interpret",))
def _run(featT_p, sidxC_p, sidxR_p, betaC_p, interpret=False):
    return pl.pallas_call(
        _body,
        grid=(3, NB),
        in_specs=[
            pl.BlockSpec((16, BH), lambda p, b: (0, b)),
            pl.BlockSpec((BH, 1), lambda p, b: (b, 0)),
            pl.BlockSpec((1, 1, BH), lambda p, b: (b, 0, 0)),
            pl.BlockSpec((BH, 1), lambda p, b: (b, 0)),
        ],
        out_specs=pl.BlockSpec((1, 1), lambda p, b: (0, 0)),
        out_shape=jax.ShapeDtypeStruct((1, 1), jnp.float32),
        scratch_shapes=[
            pltpu.VMEM((8, K), jnp.float32),
            pltpu.VMEM((16, K), jnp.float32),
            pltpu.VMEM((8, K), jnp.float32),
            pltpu.VMEM((K, 8), jnp.float32),
            pltpu.VMEM((8, BH), jnp.float32),
            pltpu.VMEM((8, BH), jnp.float32),
            pltpu.SMEM((8,), jnp.float32),
        ],
        interpret=interpret,
    )(featT_p, sidxC_p, sidxR_p, betaC_p)


def kernel(pred_beta, pred_ccoords, pred_energy, pred_pos, pred_time,
           pred_id, t_idx, t_energy, t_pos, t_time, t_pid, rowsplits):
    feat = jnp.concatenate(
        [pred_beta, pred_ccoords, pred_energy, t_energy, pred_pos, t_pos,
         t_time, pred_id], axis=1)
    pad = VP - V
    featT_p = jnp.pad(feat.T, ((0, 0), (0, pad)))
    sidx_p = jnp.pad(t_idx, ((0, pad), (0, 0)), constant_values=-1)
    sidxR_p = jnp.reshape(sidx_p, (NB, 1, BH))
    betaC_p = jnp.pad(pred_beta, ((0, pad), (0, 0)))
    loss = _run(featT_p, sidx_p, sidxR_p, betaC_p)
    return pred_beta, jnp.reshape(loss, (1,))


# speedup vs baseline: 4.5060x; 1.1824x over previous
import functools

import jax
import jax.numpy as jnp
from jax.experimental import pallas as pl
from jax.experimental.pallas import tpu as pltpu

V = 50000
K = 256
BH = 2048
NB = 25
VP = BH * NB
Q_MIN = 0.5
S_B = 1.0
_DN = (((1,), (0,)), ((), ()))


def _atanh(b):
    return 0.5 * jnp.log((1.0 + b) / (1.0 - b))


def _dot3(a, b16):
    hi = a.astype(jnp.bfloat16)
    r1 = a - hi.astype(jnp.float32)
    mid = r1.astype(jnp.bfloat16)
    lo = (r1 - mid.astype(jnp.float32)).astype(jnp.bfloat16)
    o = jax.lax.dot_general(hi, b16, _DN, preferred_element_type=jnp.float32)
    o = o + jax.lax.dot_general(mid, b16, _DN,
                                preferred_element_type=jnp.float32)
    o = o + jax.lax.dot_general(lo, b16, _DN,
                                preferred_element_type=jnp.float32)
    return o


def _body(featT_ref, sidxC_ref, sidxR_ref, betaC_ref, out_ref,
          maxb_s, sums_s, alpha_s, alphaT_s, repacc_s, sacc_s, scal_s):
    p = pl.program_id(0)
    b = pl.program_id(1)

    @pl.when(jnp.logical_and(p == 0, b == 0))
    def _init():
        maxb_s[...] = jnp.full((8, K), -1.0, jnp.float32)
        sums_s[...] = jnp.zeros((16, K), jnp.float32)
        repacc_s[...] = jnp.zeros((8, BH), jnp.float32)
        sacc_s[...] = jnp.zeros((8, BH), jnp.float32)

    def col_onehot():
        s = sidxC_ref[...]
        hid = b * BH + jax.lax.broadcasted_iota(jnp.int32, (BH, 1), 0)
        nn = jnp.logical_and(s >= 0, hid < V)
        sc = jnp.clip(s, 0, K - 1)
        kiota = jax.lax.broadcasted_iota(jnp.int32, (BH, K), 1)
        return jnp.logical_and(sc == kiota, nn)

    def row_masks():
        s = sidxR_ref[0]
        hid = b * BH + jax.lax.broadcasted_iota(jnp.int32, (1, BH), 1)
        validb = hid < V
        nnb = jnp.logical_and(s >= 0, validb)
        sc = jnp.clip(s, 0, K - 1)
        kiota = jax.lax.broadcasted_iota(jnp.int32, (K, BH), 0)
        onehotT = jnp.logical_and(sc == kiota, nnb)
        return (validb.astype(jnp.float32), nnb.astype(jnp.float32),
                onehotT.astype(jnp.bfloat16))

    @pl.when(p == 0)
    def _phase0():
        onehot = col_onehot()
        beta = jnp.clip(betaC_ref[...], 1e-4, 1.0 - 1e-4)
        cand = jnp.where(onehot, beta, -1.0)
        bm8 = jnp.max(cand.reshape(BH // 8, 8, K), axis=0)
        maxb_s[...] = jnp.maximum(maxb_s[...], bm8)

    @pl.when(jnp.logical_and(p == 1, b == 0))
    def _collapse_max():
        m = jnp.max(maxb_s[...], axis=0, keepdims=True)
        maxb_s[...] = jnp.broadcast_to(m, (8, K))

    @pl.when(p == 1)
    def _phase1():
        vm, nn, onehotT16 = row_masks()
        beta = jnp.clip(featT_ref[0:1, :], 1e-4, 1.0 - 1e-4)
        x0 = featT_ref[1:2, :]
        x1 = featT_ref[2:3, :]
        pe = featT_ref[3:4, :]
        te = featT_ref[4:5, :]
        pp0 = featT_ref[5:6, :]
        pp1 = featT_ref[6:7, :]
        tp0 = featT_ref[7:8, :]
        tp1 = featT_ref[8:9, :]
        tt = featT_ref[9:10, :]
        q = (_atanh(beta) ** 2 + Q_MIN) * tt
        pad1 = 1.0 - vm
        el = (te - pe) ** 2 / (te * te + pad1)
        pll = ((tp0 - pp0) ** 2 / (tp0 * tp0 + pad1)
               + (tp1 - pp1) ** 2 / (tp1 * tp1 + pad1))
        gmax = _dot3(maxb_s[...], onehotT16)
        maxb_hit = gmax[0:1, :]
        is_a = (beta == maxb_hit).astype(jnp.float32) * nn
        pw = beta * nn
        Wt = jnp.concatenate(
            [nn, is_a, is_a * x0, is_a * x1, is_a * q, is_a * beta,
             pw, pw * el, pw * pll, jnp.zeros((7, BH), jnp.float32)],
            axis=0)
        onehot16 = col_onehot().astype(jnp.bfloat16)
        sums_s[...] = sums_s[...] + _dot3(Wt, onehot16)
        isn = vm * (1.0 - nn)
        idv = featT_ref[10:16, :]
        idsq = jnp.sum(idv * idv, axis=0, keepdims=True)
        sacc_s[...] = sacc_s[...] + jnp.concatenate(
            [beta * isn, isn, idsq, jnp.zeros((5, BH), jnp.float32)],
            axis=0)

    @pl.when(jnp.logical_and(p == 2, b == 0))
    def _epilogue():
        nh = sums_s[0:1, :]
        den = sums_s[1:2, :] + 1e-9
        xa0 = sums_s[2:3, :] / den
        xa1 = sums_s[3:4, :] / den
        qa = sums_s[4:5, :] / den
        ba = sums_s[5:6, :] / den
        exists = (nh > 0.0).astype(jnp.float32)
        plden = sums_s[6:7, :] + 1e-9
        pl0 = sums_s[7:8, :] / plden
        pl1 = sums_s[8:9, :] / plden
        wk = qa * exists
        arows = jnp.concatenate(
            [xa0, xa1, qa, wk, jnp.zeros((4, K), jnp.float32)], axis=0)
        alpha_s[...] = arows
        alphaT_s[...] = jax.lax.transpose(arows, (1, 0))
        n_obj = jnp.sum(exists) + 1e-9
        minb = jnp.sum((1.0 - ba) * exists) / n_obj
        payload = jnp.sum((pl0 + pl1) * exists) / n_obj
        scal_s[5] = minb + payload

    @pl.when(p == 2)
    def _phase2():
        vm, nn, onehotT16 = row_masks()
        beta = jnp.clip(featT_ref[0:1, :], 1e-4, 1.0 - 1e-4)
        x0 = featT_ref[1:2, :]
        x1 = featT_ref[2:3, :]
        tt = featT_ref[9:10, :]
        q = (_atanh(beta) ** 2 + Q_MIN) * tt
        qb = q * nn
        gT = _dot3(alpha_s[...], onehotT16)
        xa0h = gT[0:1, :]
        xa1h = gT[1:2, :]
        qah = gT[2:3, :]
        wkh = gT[3:4, :]
        d2a = (x0 - xa0h) ** 2 + (x1 - xa1h) ** 2
        att_r = (qb * qah) * d2a
        hs = jnp.maximum(0.0, 1.0 - jnp.sqrt(d2a + 1e-9))
        same_r = (hs * qb) * wkh
        xa0c = alphaT_s[:, 0:1]
        xa1c = alphaT_s[:, 1:2]
        wkc = alphaT_s[:, 3:4]
        d2 = (x0 - xa0c) ** 2 + (x1 - xa1c) ** 2
        hinge = jnp.maximum(0.0, 1.0 - jnp.sqrt(d2 + 1e-9))
        repm = (hinge * qb) * wkc
        repacc_s[...] = repacc_s[...] + jnp.sum(
            repm.reshape(K // 8, 8, BH), axis=0)
        sacc_s[...] = sacc_s[...] + jnp.concatenate(
            [jnp.zeros((3, BH), jnp.float32), att_r, same_r,
             jnp.zeros((3, BH), jnp.float32)], axis=0)

        @pl.when(b == NB - 1)
        def _final():
            att = jnp.sum(sacc_s[3:4, :])
            corr = jnp.sum(sacc_s[4:5, :])
            rep = jnp.sum(repacc_s[...]) - corr
            noise_num = jnp.sum(sacc_s[0:1, :])
            noise_den = jnp.sum(sacc_s[1:2, :])
            idsq = jnp.sum(sacc_s[2:3, :])
            noise = S_B * noise_num / (noise_den + 1e-9)
            cls = 1e-8 * idsq / (V * 6.0)
            loss = att / V + rep / V + scal_s[5] + noise + cls
            out_ref[...] = jnp.reshape(loss, (1, 1))


@functools.partial(jax.jit, static_argnames=("interpret",))
def _run(featT_p, sidxC_p, sidxR_p, betaC_p, interpret=False):
    return pl.pallas_call(
        _body,
        grid=(3, NB),
        in_specs=[
            pl.BlockSpec((16, BH), lambda p, b: (0, b)),
            pl.BlockSpec((BH, 1), lambda p, b: (b, 0)),
            pl.BlockSpec((1, 1, BH), lambda p, b: (b, 0, 0)),
            pl.BlockSpec((BH, 1), lambda p, b: (b, 0)),
        ],
        out_specs=pl.BlockSpec((1, 1), lambda p, b: (0, 0)),
        out_shape=jax.ShapeDtypeStruct((1, 1), jnp.float32),
        scratch_shapes=[
            pltpu.VMEM((8, K), jnp.float32),
            pltpu.VMEM((16, K), jnp.float32),
            pltpu.VMEM((8, K), jnp.float32),
            pltpu.VMEM((K, 8), jnp.float32),
            pltpu.VMEM((8, BH), jnp.float32),
            pltpu.VMEM((8, BH), jnp.float32),
            pltpu.SMEM((8,), jnp.float32),
        ],
        interpret=interpret,
    )(featT_p, sidxC_p, sidxR_p, betaC_p)


def kernel(pred_beta, pred_ccoords, pred_energy, pred_pos, pred_time,
           pred_id, t_idx, t_energy, t_pos, t_time, t_pid, rowsplits):
    feat = jnp.concatenate(
        [pred_beta, pred_ccoords, pred_energy, t_energy, pred_pos, t_pos,
         t_time, pred_id], axis=1)
    pad = VP - V
    featT_p = jnp.pad(feat.T, ((0, 0), (0, pad)))
    sidx_p = jnp.pad(t_idx, ((0, pad), (0, 0)), constant_values=-1)
    sidxR_p = jnp.reshape(sidx_p, (NB, 1, BH))
    betaC_p = jnp.pad(pred_beta, ((0, pad), (0, 0)))
    loss = _run(featT_p, sidx_p, sidxR_p, betaC_p)
    return pred_beta, jnp.reshape(loss, (1,))
